# baseline (speedup 1.0000x reference)
import functools

import numpy as np
import jax
import jax.numpy as jnp
from jax import lax
from jax.experimental import pallas as pl
from jax.experimental.pallas import tpu as pltpu
from jax.experimental.pallas import tpu_sc as plsc

VOCAB = 100000
DIM = 1024
HALF = DIM // 2
BATCH = 4
SEQ = 2048
B = BATCH * SEQ

NC, NS = 2, 16
NW = NC * NS
CH = 32
MAXBUF = 3

SLICES = [1024, 1024]
OFFS = [0, 1024]
RB = 256

_sc_mesh = plsc.VectorSubcoreMesh(core_axis_name="c", subcore_axis_name="s")


def _make_sc_gather(pos_off, pos_len):
    rows = BATCH * pos_len
    bpw = rows // NW
    nch = bpw // CH
    nbuf = min(nch, MAXBUF)

    @functools.partial(
        pl.kernel,
        mesh=_sc_mesh,
        out_type=jax.ShapeDtypeStruct((rows, DIM), jnp.float32),
        scratch_types=[
            pltpu.VMEM((nch, CH), jnp.int32),
            [pltpu.VMEM((CH, DIM), jnp.float32) for _ in range(nbuf)],
            [pltpu.SemaphoreType.DMA for _ in range(nbuf)],
            [pltpu.SemaphoreType.DMA for _ in range(nbuf)],
        ],
    )
    def _sc_gather_slice(table_hbm, idx_hbm, out_hbm, idx_v, bufs, gsems, wsems):
        wid = lax.axis_index("s") * NC + lax.axis_index("c")
        base = wid * bpw
        pltpu.sync_copy(idx_hbm.at[wid], idx_v)
        gs = [None] * nch
        ws = [None] * nch
        for j in range(min(nbuf, nch)):
            gs[j] = pltpu.async_copy(
                table_hbm.at[idx_v.at[j]], bufs[j], gsems[j]
            )
        for j in range(nch):
            b = j % nbuf
            gs[j].wait()
            ws[j] = pltpu.async_copy(
                bufs[b], out_hbm.at[pl.ds(base + j * CH, CH)], wsems[b]
            )
            nxt = j + nbuf
            if nxt < nch:
                ws[j].wait()
                gs[nxt] = pltpu.async_copy(
                    table_hbm.at[idx_v.at[nxt]], bufs[b], gsems[b]
                )
        for j in range(max(0, nch - nbuf), nch):
            ws[j].wait()

    return _sc_gather_slice


_SC_GATHER = [_make_sc_gather(o, p) for o, p in zip(OFFS, SLICES)]


def _rope_tables():
    fi = np.arange(HALF, dtype=np.float32)
    freqs = (1.0 / (10000.0 ** (fi / DIM))).astype(np.float32)
    pos = np.arange(SEQ, dtype=np.float32)
    angles = pos[:, None] * freqs[None, :]
    return np.sin(angles), np.cos(angles)


_SIN_NP, _COS_NP = _rope_tables()


def _rope_first_body(e_ref, s_ref, c_ref, o_ref):
    xe = e_ref[:, :HALF]
    xo = e_ref[:, HALF:]
    s = s_ref[...].astype(jnp.float32)
    c = c_ref[...].astype(jnp.float32)
    o_ref[:, :HALF] = xe * c - xo * s
    o_ref[:, HALF:] = xe * s + xo * c


def _rope_chain_body(e_ref, s_ref, c_ref, prev_ref, o_ref):
    del prev_ref
    _rope_first_body(e_ref, s_ref, c_ref, o_ref)


_OUT_BLKS = SEQ // RB


def _make_rope(slice_idx):
    pos_off, pos_len = OFFS[slice_idx], SLICES[slice_idx]
    npb = pos_len // RB
    ob0 = pos_off // RB
    in_specs = [
        pl.BlockSpec((RB, DIM), lambda p, b, n=npb: (b * n + p, 0)),
        pl.BlockSpec((RB, HALF), lambda p, b, o=ob0: (o + p, 0)),
        pl.BlockSpec((RB, HALF), lambda p, b, o=ob0: (o + p, 0)),
    ]
    body = _rope_first_body
    aliases = {}
    if slice_idx > 0:
        in_specs.append(pl.BlockSpec(memory_space=pl.MemorySpace.ANY))
        body = _rope_chain_body
        aliases = {3: 0}
    return pl.pallas_call(
        body,
        grid=(npb, BATCH),
        in_specs=in_specs,
        out_specs=pl.BlockSpec(
            (RB, DIM), lambda p, b, o=ob0: (b * _OUT_BLKS + o + p, 0)
        ),
        out_shape=jax.ShapeDtypeStruct((B, DIM), jnp.float32),
        input_output_aliases=aliases,
        name=f"rope_slice_{slice_idx}",
    )


_ROPE = [_make_rope(s) for s in range(len(SLICES))]


def kernel(x, W):
    sin_t = jnp.asarray(_SIN_NP, dtype=jnp.bfloat16)
    cos_t = jnp.asarray(_COS_NP, dtype=jnp.bfloat16)
    embs = [
        g(W, x[:, o : o + p].reshape(NW, (BATCH * p) // (NW * CH), CH))
        for g, o, p in zip(_SC_GATHER, OFFS, SLICES)
    ]
    out = _ROPE[0](embs[0], sin_t, cos_t)
    for s in range(1, len(SLICES)):
        out = _ROPE[s](embs[s], sin_t, cos_t, out)
    return out.reshape(BATCH, SEQ, DIM)


# speedup vs baseline: 1.0537x; 1.0537x over previous
import functools

import numpy as np
import jax
import jax.numpy as jnp
from jax import lax
from jax.experimental import pallas as pl
from jax.experimental.pallas import tpu as pltpu
from jax.experimental.pallas import tpu_sc as plsc

VOCAB = 100000
DIM = 1024
HALF = DIM // 2
BATCH = 4
SEQ = 2048
B = BATCH * SEQ

NC, NS = 2, 16
NW = NC * NS
CH = 32
MAXBUF = 3

SLICES = [1536, 512]
OFFS = [0, 1536]
RB = 512

_sc_mesh = plsc.VectorSubcoreMesh(core_axis_name="c", subcore_axis_name="s")


def _make_sc_gather(pos_off, pos_len):
    rows = BATCH * pos_len
    bpw = rows // NW
    nch = bpw // CH
    nbuf = min(nch, MAXBUF)

    @functools.partial(
        pl.kernel,
        mesh=_sc_mesh,
        out_type=jax.ShapeDtypeStruct((rows, DIM), jnp.float32),
        scratch_types=[
            pltpu.VMEM((nch, CH), jnp.int32),
            [pltpu.VMEM((CH, DIM), jnp.float32) for _ in range(nbuf)],
            [pltpu.SemaphoreType.DMA for _ in range(nbuf)],
            [pltpu.SemaphoreType.DMA for _ in range(nbuf)],
        ],
    )
    def _sc_gather_slice(table_hbm, idx_hbm, out_hbm, idx_v, bufs, gsems, wsems):
        wid = lax.axis_index("s") * NC + lax.axis_index("c")
        base = wid * bpw
        pltpu.sync_copy(idx_hbm.at[wid], idx_v)
        gs = [None] * nch
        ws = [None] * nch
        for j in range(min(nbuf, nch)):
            gs[j] = pltpu.async_copy(
                table_hbm.at[idx_v.at[j]], bufs[j], gsems[j]
            )
        for j in range(nch):
            b = j % nbuf
            gs[j].wait()
            ws[j] = pltpu.async_copy(
                bufs[b], out_hbm.at[pl.ds(base + j * CH, CH)], wsems[b]
            )
            nxt = j + nbuf
            if nxt < nch:
                ws[j].wait()
                gs[nxt] = pltpu.async_copy(
                    table_hbm.at[idx_v.at[nxt]], bufs[b], gsems[b]
                )
        for j in range(max(0, nch - nbuf), nch):
            ws[j].wait()

    return _sc_gather_slice


_SC_GATHER = [_make_sc_gather(o, p) for o, p in zip(OFFS, SLICES)]


def _rope_tables():
    fi = np.arange(HALF, dtype=np.float32)
    freqs = (1.0 / (10000.0 ** (fi / DIM))).astype(np.float32)
    pos = np.arange(SEQ, dtype=np.float32)
    angles = pos[:, None] * freqs[None, :]
    return np.sin(angles), np.cos(angles)


_SIN_NP, _COS_NP = _rope_tables()


def _rope_first_body(e_ref, s_ref, c_ref, o_ref):
    xe = e_ref[:, :HALF]
    xo = e_ref[:, HALF:]
    s = s_ref[...]
    c = c_ref[...]
    o_ref[:, :HALF] = xe * c - xo * s
    o_ref[:, HALF:] = xe * s + xo * c


def _rope_chain_body(e_ref, s_ref, c_ref, prev_ref, o_ref):
    del prev_ref
    _rope_first_body(e_ref, s_ref, c_ref, o_ref)


_OUT_BLKS = SEQ // RB


def _make_rope(slice_idx):
    pos_off, pos_len = OFFS[slice_idx], SLICES[slice_idx]
    npb = pos_len // RB
    ob0 = pos_off // RB
    in_specs = [
        pl.BlockSpec((RB, DIM), lambda p, b, n=npb: (b * n + p, 0)),
        pl.BlockSpec((RB, HALF), lambda p, b, o=ob0: (o + p, 0)),
        pl.BlockSpec((RB, HALF), lambda p, b, o=ob0: (o + p, 0)),
    ]
    body = _rope_first_body
    aliases = {}
    if slice_idx > 0:
        in_specs.append(pl.BlockSpec(memory_space=pl.MemorySpace.ANY))
        body = _rope_chain_body
        aliases = {3: 0}
    return pl.pallas_call(
        body,
        grid=(npb, BATCH),
        in_specs=in_specs,
        out_specs=pl.BlockSpec(
            (RB, DIM), lambda p, b, o=ob0: (b * _OUT_BLKS + o + p, 0)
        ),
        out_shape=jax.ShapeDtypeStruct((B, DIM), jnp.float32),
        input_output_aliases=aliases,
        name=f"rope_slice_{slice_idx}",
    )


_ROPE = [_make_rope(s) for s in range(len(SLICES))]


def kernel(x, W):
    sin_t = jnp.asarray(_SIN_NP, dtype=jnp.float32)
    cos_t = jnp.asarray(_COS_NP, dtype=jnp.float32)
    embs = [
        g(W, x[:, o : o + p].reshape(NW, (BATCH * p) // (NW * CH), CH))
        for g, o, p in zip(_SC_GATHER, OFFS, SLICES)
    ]
    out = _ROPE[0](embs[0], sin_t, cos_t)
    for s in range(1, len(SLICES)):
        out = _ROPE[s](embs[s], sin_t, cos_t, out)
    return out.reshape(BATCH, SEQ, DIM)


# speedup vs baseline: 1.0944x; 1.0386x over previous
import functools

import numpy as np
import jax
import jax.numpy as jnp
from jax import lax
from jax.experimental import pallas as pl
from jax.experimental.pallas import tpu as pltpu
from jax.experimental.pallas import tpu_sc as plsc

VOCAB = 100000
DIM = 1024
HALF = DIM // 2
BATCH = 4
SEQ = 2048
B = BATCH * SEQ

NC, NS = 2, 16
NW = NC * NS
NSLICE = 2
PSL = SEQ // NSLICE
SL = BATCH * PSL
B_PER_W = SL // NW
CH = 32
NCH = B_PER_W // CH
NBUF = min(NCH, 3)

_sc_mesh = plsc.VectorSubcoreMesh(core_axis_name="c", subcore_axis_name="s")

_WPB = PSL // B_PER_W


def _make_sc_gather(slice_idx):
    @functools.partial(
        pl.kernel,
        mesh=_sc_mesh,
        out_type=jax.ShapeDtypeStruct((SL, DIM), jnp.float32),
        scratch_types=[
            pltpu.VMEM((B_PER_W,), jnp.int32),
            [pltpu.VMEM((CH, DIM), jnp.float32) for _ in range(NBUF)],
            [pltpu.SemaphoreType.DMA for _ in range(NBUF)],
            [pltpu.SemaphoreType.DMA for _ in range(NBUF)],
        ],
    )
    def _sc_gather_slice(table_hbm, x_hbm, out_hbm, idx_v, bufs, gsems, wsems):
        wid = lax.axis_index("s") * NC + lax.axis_index("c")
        base = wid * B_PER_W
        brow = wid // _WPB
        col0 = (wid % _WPB) * B_PER_W + slice_idx * PSL
        pltpu.sync_copy(x_hbm.at[brow, pl.ds(col0, B_PER_W)], idx_v)
        gs = [None] * NCH
        ws = [None] * NCH
        for j in range(min(NBUF, NCH)):
            gs[j] = pltpu.async_copy(
                table_hbm.at[idx_v.at[pl.ds(j * CH, CH)]], bufs[j], gsems[j]
            )
        for j in range(NCH):
            b = j % NBUF
            gs[j].wait()
            ws[j] = pltpu.async_copy(
                bufs[b], out_hbm.at[pl.ds(base + j * CH, CH)], wsems[b]
            )
            nxt = j + NBUF
            if nxt < NCH:
                ws[j].wait()
                gs[nxt] = pltpu.async_copy(
                    table_hbm.at[idx_v.at[pl.ds(nxt * CH, CH)]], bufs[b], gsems[b]
                )
        for j in range(max(0, NCH - NBUF), NCH):
            ws[j].wait()

    return _sc_gather_slice


_SC_GATHER = [_make_sc_gather(s) for s in range(NSLICE)]


def _rope_tables():
    fi = np.arange(HALF, dtype=np.float32)
    freqs = (1.0 / (10000.0 ** (fi / DIM))).astype(np.float32)
    pos = np.arange(SEQ, dtype=np.float32)
    angles = pos[:, None] * freqs[None, :]
    return np.sin(angles).astype(np.float32), np.cos(angles).astype(np.float32)


_SIN, _COS = _rope_tables()


def _rope_first_body(e_ref, s_ref, c_ref, o_ref):
    xe = e_ref[:, :HALF]
    xo = e_ref[:, HALF:]
    s = s_ref[...]
    c = c_ref[...]
    o_ref[:, :HALF] = xe * c - xo * s
    o_ref[:, HALF:] = xe * s + xo * c


def _rope_chain_body(e_ref, s_ref, c_ref, prev_ref, o_ref):
    del prev_ref
    _rope_first_body(e_ref, s_ref, c_ref, o_ref)


_OUT_BLKS = SEQ // PSL


def _make_rope(slice_idx):
    in_specs = [
        pl.BlockSpec((PSL, DIM), lambda b: (b, 0)),
        pl.BlockSpec((PSL, HALF), lambda b, s=slice_idx: (s, 0)),
        pl.BlockSpec((PSL, HALF), lambda b, s=slice_idx: (s, 0)),
    ]
    body = _rope_first_body
    aliases = {}
    if slice_idx > 0:
        in_specs.append(pl.BlockSpec(memory_space=pl.MemorySpace.ANY))
        body = _rope_chain_body
        aliases = {3: 0}
    return pl.pallas_call(
        body,
        grid=(BATCH,),
        in_specs=in_specs,
        out_specs=pl.BlockSpec(
            (PSL, DIM), lambda b, s=slice_idx: (b * _OUT_BLKS + s, 0)
        ),
        out_shape=jax.ShapeDtypeStruct((B, DIM), jnp.float32),
        input_output_aliases=aliases,
        name=f"rope_slice_{slice_idx}",
    )


_ROPE = [_make_rope(s) for s in range(NSLICE)]


def kernel(x, W):
    sin_t = jnp.asarray(_SIN)
    cos_t = jnp.asarray(_COS)
    embs = [_SC_GATHER[s](W, x) for s in range(NSLICE)]
    out = _ROPE[0](embs[0], sin_t, cos_t)
    for s in range(1, NSLICE):
        out = _ROPE[s](embs[s], sin_t, cos_t, out)
    return out.reshape(BATCH, SEQ, DIM)


# speedup vs baseline: 1.1088x; 1.0131x over previous
import functools

import numpy as np
import jax
import jax.numpy as jnp
from jax import lax
from jax.experimental import pallas as pl
from jax.experimental.pallas import tpu as pltpu
from jax.experimental.pallas import tpu_sc as plsc

VOCAB = 100000
DIM = 1024
HALF = DIM // 2
BATCH = 4
SEQ = 2048
B = BATCH * SEQ

NC, NS = 2, 16
NW = NC * NS
NSLICE = 2
PSL = SEQ // NSLICE
SL = BATCH * PSL
B_PER_W = SL // NW
CH = 32
NCH = B_PER_W // CH
NBUF = min(NCH, 3)

_sc_mesh = plsc.VectorSubcoreMesh(core_axis_name="c", subcore_axis_name="s")

_WPB = PSL // B_PER_W


def _make_sc_gather(slice_idx):
    @functools.partial(
        pl.kernel,
        mesh=_sc_mesh,
        out_type=jax.ShapeDtypeStruct((SL, DIM), jnp.float32),
        scratch_types=[
            pltpu.VMEM((B_PER_W,), jnp.int32),
            [pltpu.VMEM((CH, DIM), jnp.float32) for _ in range(NBUF)],
            [pltpu.SemaphoreType.DMA for _ in range(NBUF)],
            [pltpu.SemaphoreType.DMA for _ in range(NBUF)],
        ],
    )
    def _sc_gather_slice(table_hbm, x_hbm, out_hbm, idx_v, bufs, gsems, wsems):
        wid = lax.axis_index("s") * NC + lax.axis_index("c")
        base = wid * B_PER_W
        brow = wid // _WPB
        col0 = (wid % _WPB) * B_PER_W + slice_idx * PSL
        pltpu.sync_copy(x_hbm.at[brow, pl.ds(col0, B_PER_W)], idx_v)
        gs = [None] * NCH
        ws = [None] * NCH
        for j in range(min(NBUF, NCH)):
            gs[j] = pltpu.async_copy(
                table_hbm.at[idx_v.at[pl.ds(j * CH, CH)]], bufs[j], gsems[j]
            )
        for j in range(NCH):
            b = j % NBUF
            gs[j].wait()
            ws[j] = pltpu.async_copy(
                bufs[b], out_hbm.at[pl.ds(base + j * CH, CH)], wsems[b]
            )
            nxt = j + NBUF
            if nxt < NCH:
                ws[j].wait()
                gs[nxt] = pltpu.async_copy(
                    table_hbm.at[idx_v.at[pl.ds(nxt * CH, CH)]], bufs[b], gsems[b]
                )
        for j in range(max(0, NCH - NBUF), NCH):
            ws[j].wait()

    return _sc_gather_slice


_SC_GATHER = [_make_sc_gather(s) for s in range(NSLICE)]


def _rope_tables():
    fi = np.arange(HALF, dtype=np.float32)
    freqs = (1.0 / (10000.0 ** (fi / DIM))).astype(np.float32)
    pos = np.arange(SEQ, dtype=np.float32)
    angles = pos[:, None] * freqs[None, :]
    return np.sin(angles).astype(np.float32), np.cos(angles).astype(np.float32)


_SIN, _COS = _rope_tables()


def _rope_first_body(e_ref, s_ref, c_ref, o_ref):
    xe = e_ref[:, :HALF]
    xo = e_ref[:, HALF:]
    s = s_ref[...].astype(jnp.float32)
    c = c_ref[...].astype(jnp.float32)
    o_ref[:, :HALF] = xe * c - xo * s
    o_ref[:, HALF:] = xe * s + xo * c


def _rope_chain_body(e_ref, s_ref, c_ref, prev_ref, o_ref):
    del prev_ref
    _rope_first_body(e_ref, s_ref, c_ref, o_ref)


_OUT_BLKS = SEQ // PSL


def _make_rope(slice_idx):
    in_specs = [
        pl.BlockSpec((PSL, DIM), lambda b: (b, 0)),
        pl.BlockSpec((PSL, HALF), lambda b, s=slice_idx: (s, 0)),
        pl.BlockSpec((PSL, HALF), lambda b, s=slice_idx: (s, 0)),
    ]
    body = _rope_first_body
    aliases = {}
    if slice_idx > 0:
        in_specs.append(pl.BlockSpec(memory_space=pl.MemorySpace.ANY))
        body = _rope_chain_body
        aliases = {3: 0}
    return pl.pallas_call(
        body,
        grid=(BATCH,),
        in_specs=in_specs,
        out_specs=pl.BlockSpec(
            (PSL, DIM), lambda b, s=slice_idx: (b * _OUT_BLKS + s, 0)
        ),
        out_shape=jax.ShapeDtypeStruct((B, DIM), jnp.float32),
        input_output_aliases=aliases,
        name=f"rope_slice_{slice_idx}",
    )


_ROPE = [_make_rope(s) for s in range(NSLICE)]


def kernel(x, W):
    sin_t = jnp.asarray(_SIN, dtype=jnp.bfloat16)
    cos_t = jnp.asarray(_COS, dtype=jnp.bfloat16)
    embs = [_SC_GATHER[s](W, x) for s in range(NSLICE)]
    out = _ROPE[0](embs[0], sin_t, cos_t)
    for s in range(1, NSLICE):
        out = _ROPE[s](embs[s], sin_t, cos_t, out)
    return out.reshape(BATCH, SEQ, DIM)


# speedup vs baseline: 1.1210x; 1.0110x over previous
import functools

import numpy as np
import jax
import jax.numpy as jnp
from jax import lax
from jax.experimental import pallas as pl
from jax.experimental.pallas import tpu as pltpu
from jax.experimental.pallas import tpu_sc as plsc

VOCAB = 100000
DIM = 1024
HALF = DIM // 2
BATCH = 4
SEQ = 2048
B = BATCH * SEQ

NC, NS = 2, 16
NW = NC * NS
NSLICE = 2
PSL = SEQ // NSLICE
SL = BATCH * PSL
B_PER_W = SL // NW
CH = 16
NCH = B_PER_W // CH
NBUF = min(NCH, 6)

_sc_mesh = plsc.VectorSubcoreMesh(core_axis_name="c", subcore_axis_name="s")

_WPB = PSL // B_PER_W


def _make_sc_gather(slice_idx):
    @functools.partial(
        pl.kernel,
        mesh=_sc_mesh,
        out_type=jax.ShapeDtypeStruct((SL, DIM), jnp.float32),
        scratch_types=[
            pltpu.VMEM((B_PER_W,), jnp.int32),
            [pltpu.VMEM((CH, DIM), jnp.float32) for _ in range(NBUF)],
            [pltpu.SemaphoreType.DMA for _ in range(NBUF)],
            [pltpu.SemaphoreType.DMA for _ in range(NBUF)],
        ],
    )
    def _sc_gather_slice(table_hbm, x_hbm, out_hbm, idx_v, bufs, gsems, wsems):
        wid = lax.axis_index("s") * NC + lax.axis_index("c")
        base = wid * B_PER_W
        brow = wid // _WPB
        col0 = (wid % _WPB) * B_PER_W + slice_idx * PSL
        pltpu.sync_copy(x_hbm.at[brow, pl.ds(col0, B_PER_W)], idx_v)
        gs = [None] * NCH
        ws = [None] * NCH
        for j in range(min(NBUF, NCH)):
            gs[j] = pltpu.async_copy(
                table_hbm.at[idx_v.at[pl.ds(j * CH, CH)]], bufs[j], gsems[j]
            )
        for j in range(NCH):
            b = j % NBUF
            gs[j].wait()
            ws[j] = pltpu.async_copy(
                bufs[b], out_hbm.at[pl.ds(base + j * CH, CH)], wsems[b]
            )
            nxt = j + NBUF
            if nxt < NCH:
                ws[j].wait()
                gs[nxt] = pltpu.async_copy(
                    table_hbm.at[idx_v.at[pl.ds(nxt * CH, CH)]], bufs[b], gsems[b]
                )
        for j in range(max(0, NCH - NBUF), NCH):
            ws[j].wait()

    return _sc_gather_slice


_SC_GATHER = [_make_sc_gather(s) for s in range(NSLICE)]


def _rope_tables():
    fi = np.arange(HALF, dtype=np.float32)
    freqs = (1.0 / (10000.0 ** (fi / DIM))).astype(np.float32)
    pos = np.arange(SEQ, dtype=np.float32)
    angles = pos[:, None] * freqs[None, :]
    return np.sin(angles).astype(np.float32), np.cos(angles).astype(np.float32)


_SIN, _COS = _rope_tables()


def _rope_first_body(e_ref, s_ref, c_ref, o_ref):
    xe = e_ref[:, :HALF]
    xo = e_ref[:, HALF:]
    s = s_ref[...].astype(jnp.float32)
    c = c_ref[...].astype(jnp.float32)
    o_ref[:, :HALF] = xe * c - xo * s
    o_ref[:, HALF:] = xe * s + xo * c


def _rope_chain_body(e_ref, s_ref, c_ref, prev_ref, o_ref):
    del prev_ref
    _rope_first_body(e_ref, s_ref, c_ref, o_ref)


_OUT_BLKS = SEQ // PSL


def _make_rope(slice_idx):
    in_specs = [
        pl.BlockSpec((PSL, DIM), lambda b: (b, 0)),
        pl.BlockSpec((PSL, HALF), lambda b, s=slice_idx: (s, 0)),
        pl.BlockSpec((PSL, HALF), lambda b, s=slice_idx: (s, 0)),
    ]
    body = _rope_first_body
    aliases = {}
    if slice_idx > 0:
        in_specs.append(pl.BlockSpec(memory_space=pl.MemorySpace.ANY))
        body = _rope_chain_body
        aliases = {3: 0}
    return pl.pallas_call(
        body,
        grid=(BATCH,),
        in_specs=in_specs,
        out_specs=pl.BlockSpec(
            (PSL, DIM), lambda b, s=slice_idx: (b * _OUT_BLKS + s, 0)
        ),
        out_shape=jax.ShapeDtypeStruct((B, DIM), jnp.float32),
        input_output_aliases=aliases,
        name=f"rope_slice_{slice_idx}",
    )


_ROPE = [_make_rope(s) for s in range(NSLICE)]


def kernel(x, W):
    sin_t = jnp.asarray(_SIN, dtype=jnp.bfloat16)
    cos_t = jnp.asarray(_COS, dtype=jnp.bfloat16)
    embs = [_SC_GATHER[s](W, x) for s in range(NSLICE)]
    out = _ROPE[0](embs[0], sin_t, cos_t)
    for s in range(1, NSLICE):
        out = _ROPE[s](embs[s], sin_t, cos_t, out)
    return out.reshape(BATCH, SEQ, DIM)
